# baseline (speedup 1.0000x reference)
import numpy as np
import jax
import jax.numpy as jnp
from jax.experimental import pallas as pl
from jax.experimental.pallas import tpu as pltpu
from jax.experimental.pallas import tpu_sc as plsc

N = 10000
DEG = 16
NFEAT = 512
NCLASS = 16
KCAPS = 8
DD = 16
D = 128
ROUTIT = 3
NUM_SAMP = 10 * DEG

_BLK_PRE = 2000
_BLK_RT = 2000

_B_CAP = np.kron(np.eye(KCAPS, dtype=np.float32),
                 np.ones((DD, DD), dtype=np.float32))

_F_SEL = np.tile(np.eye(DD, dtype=np.float32), (KCAPS, 1))

_FI = np.array([1724, 1411, 3219, 4281, 2350, 4219, 9180, 7029, 68, 6211, 1676, 3861, 6482, 9627, 7891, 7264, 6819, 5847, 6683, 6299, 2973, 7745, 3728, 6163, 5575, 4637, 8847, 5940, 1292, 5769, 1716, 5587, 3520, 5607, 4739, 8396, 1300, 233, 2225, 2381, 7534, 8192, 4954, 9938, 984, 7161, 9962, 4968, 2369, 8816, 2903, 8335, 995, 9671, 4147, 8032, 3079, 7869, 2024, 7453, 9480, 7283, 4232, 4252, 6094, 5708, 1965, 5767, 2829, 9076, 26, 984, 6411, 5822, 3420, 827, 7172, 4476, 9543, 802, 5686, 2348, 850, 5229, 2827, 6789, 1044, 2629, 1115, 6479, 4338, 229, 4285, 8060, 267, 4289, 7680, 9661, 1233, 989, 4717, 7341, 1673, 1250, 919, 2578, 9695, 3507, 9666, 4219, 7170, 9055, 6570, 9596, 6651, 5838, 1406, 9417, 8380, 9376, 5237, 868, 9640, 8470, 1109, 1387, 9323, 4687, 4288, 616, 4207, 978, 1498, 5207, 5457, 8836, 3435, 7936, 8526, 7267, 2718, 2362, 4729, 3555, 8230, 1181, 8139, 8643, 9713, 9981, 5771, 3818, 4710, 4540, 8573, 7284, 7687, 352, 1779, 6203], dtype=np.int32)
_FL = np.array([5, 0, 0, 4, 3, 4, 1, 2, 4, 7, 4, 4, 0, 7, 1, 4, 1, 6, 1, 3, 7, 6, 2, 4, 2, 4, 6, 6, 2, 1, 2, 6, 4, 3, 3, 0, 2, 0, 2, 6, 2, 3, 6, 7, 1, 4, 1, 2, 4, 1, 7, 4, 5, 4, 0, 6, 1, 4, 4, 0, 5, 7, 7, 1, 0, 3, 5, 6, 4, 6, 4, 2, 4, 0, 2, 7, 1, 1, 0, 6, 2, 7, 1, 2, 5, 6, 5, 7, 1, 2, 1, 5, 3, 1, 1, 0, 6, 2, 7, 6, 4, 7, 4, 4, 1, 5, 7, 6, 1, 0, 4, 3, 6, 1, 6, 2, 4, 0, 1, 1, 3, 2, 1, 1, 7, 2, 4, 1, 2, 7, 3, 4, 2, 7, 2, 6, 5, 2, 6, 7, 3, 0, 2, 5, 3, 3, 0, 5, 0, 0, 6, 4, 4, 1, 6, 1, 7, 1, 3, 4], dtype=np.int32)
_RI = np.array([6417, 669, 5307, 7936, 1565, 2514, 5651, 7465, 4562, 3971, 7945, 1207, 2520, 7611, 318, 5618, 9724, 5746, 9389, 1637, 7653, 8372, 167, 4090, 3013, 357, 693, 5668, 8183, 6997, 7333, 2566, 4646, 2636, 9480, 5752, 157, 9721, 2584, 4732, 254, 1994, 2550, 3224, 7391, 1895, 4131, 426, 7330, 6961, 4319, 9821, 6380, 8771, 3323, 5968, 6462, 6015, 7178, 3430, 6732, 3417, 8879, 8842, 8634, 7623, 1348, 3497, 4026, 6147, 3103, 6515, 8582, 6590, 7215, 4145, 3291, 2965, 3982, 237, 7633, 7539, 7552, 5711, 5890, 682, 5909, 2349, 8126, 3965, 5093, 4957, 6059, 9744, 8061, 8964, 6338, 5058, 6625, 5833, 4193, 1044, 3311, 6274, 6147, 7947, 7617, 4690, 9728, 3167, 8343, 6996, 6656, 5584, 9381, 1412, 3634, 917, 1671, 6858, 5240, 1819, 4888, 5800, 7807, 8741, 3400, 242, 3307, 7869, 6945, 5022, 242, 3954, 699, 6119, 2227, 5925, 7139, 8273, 8393, 5834, 9138, 2122, 2043, 4119, 7386, 2944, 1822, 8753, 683, 2396, 9592, 3793, 6616, 4477, 9162, 6754, 5724, 8680], dtype=np.int32)
_RL = np.array([6, 4, 1, 4, 7, 4, 0, 7, 3, 6, 2, 4, 3, 4, 5, 1, 5, 2, 2, 1, 1, 6, 4, 2, 1, 1, 4, 5, 0, 7, 7, 7, 2, 0, 6, 0, 1, 2, 7, 0, 0, 0, 3, 7, 3, 3, 7, 4, 0, 3, 7, 2, 5, 2, 5, 2, 7, 4, 1, 0, 5, 7, 3, 3, 4, 5, 0, 0, 0, 1, 7, 6, 5, 5, 5, 3, 0, 7, 1, 1, 3, 3, 1, 1, 7, 6, 5, 1, 7, 3, 0, 4, 3, 7, 4, 5, 0, 6, 7, 2, 5, 5, 4, 5, 0, 3, 5, 2, 4, 1, 3, 4, 1, 0, 0, 5, 6, 6, 7, 2, 6, 5, 6, 7, 4, 4, 5, 4, 7, 7, 3, 0, 5, 3, 5, 5, 5, 7, 7, 6, 4, 6, 0, 5, 0, 1, 7, 4, 4, 1, 1, 6, 5, 2, 5, 5, 1, 7, 3, 5], dtype=np.int32)


def _norm_caps(v, bcap):
    ss = jnp.dot(v * v, bcap, preferred_element_type=jnp.float32)
    return v / jnp.maximum(jnp.sqrt(ss), 1e-12)


def _pre_body(x_ref, w_ref, b_ref, fcw_ref, fcb_ref, bcap_ref,
              h0_ref, x1_ref):
    bcap = bcap_ref[...]
    h = jnp.dot(x_ref[...], w_ref[...], preferred_element_type=jnp.float32)
    h = jnp.maximum(h + b_ref[...], 0.0)
    h0_ref[...] = h
    xs = _norm_caps(h, bcap)
    t = jnp.dot(xs, fcw_ref[...], preferred_element_type=jnp.float32)
    t = jnp.maximum(t + fcb_ref[...], 0.0)
    x1_ref[...] = _norm_caps(t, bcap)


def _pre_call(x, w, b, fcw_bd, fcb, bcap):
    return pl.pallas_call(
        _pre_body,
        grid=(N // _BLK_PRE,),
        in_specs=[
            pl.BlockSpec((_BLK_PRE, NFEAT), lambda i: (i, 0)),
            pl.BlockSpec((NFEAT, D), lambda i: (0, 0)),
            pl.BlockSpec((1, D), lambda i: (0, 0)),
            pl.BlockSpec((D, D), lambda i: (0, 0)),
            pl.BlockSpec((1, D), lambda i: (0, 0)),
            pl.BlockSpec((D, D), lambda i: (0, 0)),
        ],
        out_specs=[
            pl.BlockSpec((_BLK_PRE, D), lambda i: (i, 0)),
            pl.BlockSpec((_BLK_PRE, D), lambda i: (i, 0)),
        ],
        out_shape=[
            jax.ShapeDtypeStruct((N, D), jnp.float32),
            jax.ShapeDtypeStruct((N, D), jnp.float32),
        ],
    )(x, w, b, fcw_bd, fcb, bcap)


def _sc_gather(xsrc, idx_flat):
    mesh = plsc.VectorSubcoreMesh(core_axis_name="c", subcore_axis_name="s")
    total = idx_flat.shape[1]
    win = 128 if total % 128 == 0 else total
    width = xsrc.shape[1]

    @pl.kernel(out_type=jax.ShapeDtypeStruct((total, width), xsrc.dtype),
               mesh=mesh)
    def gk(x_hbm, i_hbm, o_hbm):
        def body(i_vmem, o_vmem):
            pltpu.sync_copy(x_hbm.at[i_vmem.at[0]], o_vmem)

        pltpu.emit_pipeline(
            body,
            grid=(total // win,),
            in_specs=[pl.BlockSpec((1, win), index_map=lambda i: (0, i))],
            out_specs=[pl.BlockSpec((win, width), index_map=lambda i: (i, 0))],
            core_axis_name=("c", "s"),
            dimension_semantics=(pltpu.PARALLEL,),
        )(i_hbm, o_hbm)

    return gk(xsrc, idx_flat)


def _route_iters(zs, u0, g, bcap):
    u = u0
    ub = u0
    one_m = 1.0 / DEG
    one_k = 1.0 / KCAPS

    def tree_sum(vals):
        while len(vals) > 1:
            vals = [a + b for a, b in zip(vals[::2], vals[1::2])]
        return vals[0]

    for it in range(ROUTIT):
        if it == 0:
            w0 = g * one_m + (1.0 - g) * one_k
            u = tree_sum(list(zs)) * w0 + ub
        else:
            s = None
            a1 = None
            a2 = None
            for m in range(DEG):
                p = jnp.dot(zs[m] * u, bcap,
                            preferred_element_type=jnp.float32)
                e = jnp.exp(p)
                ts = jnp.sum(e, axis=1, keepdims=True) * (1.0 / DD)
                ze = zs[m] * e
                zet = ze / ts
                s = e if s is None else s + e
                a1 = ze if a1 is None else a1 + ze
                a2 = zet if a2 is None else a2 + zet
            u = (g * a1 / s + (1.0 - g) * a2) + ub
        ub = u
        if it < ROUTIT - 1:
            u = _norm_caps(u, bcap)
    return u


def _route1_body(z_ref, x_ref, g_ref, bcap_ref, buf_ref, out_ref):
    del buf_ref
    bcap = bcap_ref[...]
    zs = [z_ref[m] for m in range(DEG)]
    u = _route_iters(zs, x_ref[...], g_ref[...], bcap)
    out_ref[...] = _norm_caps(jnp.maximum(u, 0.0), bcap)


def _route2_body(z_ref, x_ref, g_ref, bcap_ref, mlpw_ref, mlpb_ref,
                 bufh_ref, buflp_ref, h_ref, logp_ref):
    del bufh_ref, buflp_ref
    bcap = bcap_ref[...]
    zs = [z_ref[m] for m in range(DEG)]
    u = _route_iters(zs, x_ref[...], g_ref[...], bcap)
    h_ref[...] = u
    logit = jnp.dot(u, mlpw_ref[...], preferred_element_type=jnp.float32)
    logit = logit + mlpb_ref[...]
    mx = jnp.max(logit, axis=1, keepdims=True)
    lse = jnp.log(jnp.sum(jnp.exp(logit - mx), axis=1, keepdims=True)) + mx
    logp_ref[...] = logit - lse


def _route1_call(z, x1, g, bcap, buf, off):
    nblk = z.shape[1] // _BLK_RT
    return pl.pallas_call(
        _route1_body,
        grid=(nblk,),
        in_specs=[
            pl.BlockSpec((DEG, _BLK_RT, D), lambda i: (0, i, 0)),
            pl.BlockSpec((_BLK_RT, D), lambda i: (i + off, 0)),
            pl.BlockSpec((1, D), lambda i: (0, 0)),
            pl.BlockSpec((D, D), lambda i: (0, 0)),
            pl.BlockSpec(memory_space=pltpu.MemorySpace.HBM),
        ],
        out_specs=pl.BlockSpec((_BLK_RT, D), lambda i: (i + off, 0)),
        out_shape=jax.ShapeDtypeStruct((N, D), jnp.float32),
        input_output_aliases={4: 0},
    )(z, x1, g, bcap, buf)


def _route2_call(z, x2, g, bcap, mlpw, mlpb, bufh, buflp, off):
    nblk = z.shape[1] // _BLK_RT
    return pl.pallas_call(
        _route2_body,
        grid=(nblk,),
        in_specs=[
            pl.BlockSpec((DEG, _BLK_RT, D), lambda i: (0, i, 0)),
            pl.BlockSpec((_BLK_RT, D), lambda i: (i + off, 0)),
            pl.BlockSpec((1, D), lambda i: (0, 0)),
            pl.BlockSpec((D, D), lambda i: (0, 0)),
            pl.BlockSpec((D, NCLASS), lambda i: (0, 0)),
            pl.BlockSpec((1, NCLASS), lambda i: (0, 0)),
            pl.BlockSpec(memory_space=pltpu.MemorySpace.HBM),
            pl.BlockSpec(memory_space=pltpu.MemorySpace.HBM),
        ],
        out_specs=[
            pl.BlockSpec((_BLK_RT, D), lambda i: (i + off, 0)),
            pl.BlockSpec((_BLK_RT, NCLASS), lambda i: (i + off, 0)),
        ],
        out_shape=[
            jax.ShapeDtypeStruct((N, D), jnp.float32),
            jax.ShapeDtypeStruct((N, NCLASS), jnp.float32),
        ],
        input_output_aliases={6: 0, 7: 1},
    )(z, x2, g, bcap, mlpw, mlpb, bufh, buflp)


def _softplus(v):
    return jnp.maximum(v, 0.0) + jnp.log(1.0 + jnp.exp(-jnp.abs(v)))


def _disc_body(fake_ref, real_ref, fl_ref, rl_ref, fsel_ref, dw_ref, db_ref,
               c1w_ref, c1b_ref, c2w_ref, c2b_ref, g_ref, d_ref):
    def pick(rows, lab):
        ci = jax.lax.shift_right_logical(
            jax.lax.broadcasted_iota(jnp.int32, rows.shape, 1), 4)
        masked = jnp.where(ci == lab, rows, 0.0)
        return jnp.dot(masked, fsel_ref[...],
                       preferred_element_type=jnp.float32)

    def disc(sv):
        t = jnp.dot(sv, dw_ref[...], preferred_element_type=jnp.float32)
        t = jnp.maximum(t + db_ref[...], 0.0)
        lg1 = jnp.dot(t, c1w_ref[...],
                      preferred_element_type=jnp.float32) + c1b_ref[...]
        lg2 = jnp.dot(t, c2w_ref[...],
                      preferred_element_type=jnp.float32) + c2b_ref[...]
        return lg1, lg2

    def clsloss(lg, lb):
        mx = jnp.max(lg, axis=1, keepdims=True)
        lse = jnp.log(jnp.sum(jnp.exp(lg - mx), axis=1, keepdims=True)) + mx
        oh = jax.lax.broadcasted_iota(jnp.int32, lg.shape, 1) == lb
        sel = jnp.sum(jnp.where(oh, lg, 0.0), axis=1, keepdims=True)
        return jnp.mean(lse - sel)

    d_fake, prob_fake = disc(pick(fake_ref[...], fl_ref[...]))
    d_real, prob_real = disc(pick(real_ref[...], rl_ref[...]))
    g_total = jnp.mean(_softplus(-d_fake)) + clsloss(prob_real, rl_ref[...])
    d_total = (jnp.mean(_softplus(-d_real)) + jnp.mean(_softplus(d_fake))
               + clsloss(prob_fake, fl_ref[...]))
    g_ref[...] = g_total.reshape(1, 1)
    d_ref[...] = d_total.reshape(1, 1)


def _disc_call(fake, real, fl, rl, fsel, dw, db, c1w, c1b, c2w, c2b):
    def full(s):
        return pl.BlockSpec(s, lambda: (0,) * len(s))
    return pl.pallas_call(
        _disc_body,
        in_specs=[
            full((NUM_SAMP, D)), full((NUM_SAMP, D)),
            full((NUM_SAMP, 1)), full((NUM_SAMP, 1)),
            full((D, DD)),
            full((DD, KCAPS)), full((1, KCAPS)),
            full((KCAPS, 1)), full((1, 1)),
            full((KCAPS, KCAPS)), full((1, KCAPS)),
        ],
        out_specs=[full((1, 1)), full((1, 1))],
        out_shape=[
            jax.ShapeDtypeStruct((1, 1), jnp.float32),
            jax.ShapeDtypeStruct((1, 1), jnp.float32),
        ],
    )(fake, real, fl, rl, fsel, dw, db, c1w, c1b, c2w, c2b)


def kernel(x, nb, params):
    bcap = jnp.asarray(_B_CAP)

    fcw = params['fc_W']
    fcw_bd = jnp.zeros((D, D), jnp.float32)
    for k in range(KCAPS):
        fcw_bd = fcw_bd.at[k * DD:(k + 1) * DD, k * DD:(k + 1) * DD].set(fcw[k])
    fcb = params['fc_b'].reshape(1, D)

    gate = jax.nn.sigmoid(params['param']).reshape(1, 1)
    g_vec = jnp.broadcast_to(gate, (1, D)).astype(jnp.float32)

    h0, x1 = _pre_call(x, params['pca_W'], params['pca_b'].reshape(1, D),
                       fcw_bd, fcb, bcap)

    nchunk = 5
    chn = N // nchunk
    idx_mm = nb.T.astype(jnp.int32)

    nblk_per_chunk = chn // _BLK_RT

    def chunk_idx(c):
        return idx_mm[:, c * chn:(c + 1) * chn].reshape(1, DEG * chn)

    x2 = jnp.zeros((N, D), jnp.float32)
    for c in range(nchunk):
        zc = _sc_gather(x1, chunk_idx(c)).reshape(DEG, chn, D)
        x2 = _route1_call(zc, x1, g_vec, bcap, x2, c * nblk_per_chunk)

    fi_pad = jnp.asarray(np.pad(_FI, (0, 256 - NUM_SAMP)).reshape(1, 256))
    ri_pad = jnp.asarray(np.pad(_RI, (0, 256 - NUM_SAMP)).reshape(1, 256))
    fake_rows = _sc_gather(h0, fi_pad)[:NUM_SAMP]

    h = jnp.zeros((N, D), jnp.float32)
    logp = jnp.zeros((N, NCLASS), jnp.float32)
    for c in range(nchunk):
        zc = _sc_gather(x2, chunk_idx(c)).reshape(DEG, chn, D)
        h, logp = _route2_call(zc, x2, g_vec, bcap, params['mlp_W'],
                               params['mlp_b'].reshape(1, NCLASS),
                               h, logp, c * nblk_per_chunk)

    real_rows = _sc_gather(h, ri_pad)[:NUM_SAMP]

    g_t, d_t = _disc_call(
        fake_rows, real_rows,
        jnp.asarray(_FL).reshape(NUM_SAMP, 1),
        jnp.asarray(_RL).reshape(NUM_SAMP, 1),
        jnp.asarray(_F_SEL),
        params['disc_W'], params['disc_b'].reshape(1, KCAPS),
        params['cls1_W'], params['cls1_b'].reshape(1, 1),
        params['cls2_W'], params['cls2_b'].reshape(1, KCAPS))

    return logp, g_t.reshape(()), d_t.reshape(()), h


# speedup vs baseline: 1.2157x; 1.2157x over previous
import numpy as np
import jax
import jax.numpy as jnp
from jax.experimental import pallas as pl
from jax.experimental.pallas import tpu as pltpu
from jax.experimental.pallas import tpu_sc as plsc

N = 10000
DEG = 16
NFEAT = 512
NCLASS = 16
KCAPS = 8
DD = 16
D = 128
ROUTIT = 3
NUM_SAMP = 10 * DEG

_BLK_PRE = 2000
_BLK_RT = 1000

_B_CAP = np.kron(np.eye(KCAPS, dtype=np.float32),
                 np.ones((DD, DD), dtype=np.float32))

_F_SEL = np.tile(np.eye(DD, dtype=np.float32), (KCAPS, 1))

_FI = np.array([1724, 1411, 3219, 4281, 2350, 4219, 9180, 7029, 68, 6211, 1676, 3861, 6482, 9627, 7891, 7264, 6819, 5847, 6683, 6299, 2973, 7745, 3728, 6163, 5575, 4637, 8847, 5940, 1292, 5769, 1716, 5587, 3520, 5607, 4739, 8396, 1300, 233, 2225, 2381, 7534, 8192, 4954, 9938, 984, 7161, 9962, 4968, 2369, 8816, 2903, 8335, 995, 9671, 4147, 8032, 3079, 7869, 2024, 7453, 9480, 7283, 4232, 4252, 6094, 5708, 1965, 5767, 2829, 9076, 26, 984, 6411, 5822, 3420, 827, 7172, 4476, 9543, 802, 5686, 2348, 850, 5229, 2827, 6789, 1044, 2629, 1115, 6479, 4338, 229, 4285, 8060, 267, 4289, 7680, 9661, 1233, 989, 4717, 7341, 1673, 1250, 919, 2578, 9695, 3507, 9666, 4219, 7170, 9055, 6570, 9596, 6651, 5838, 1406, 9417, 8380, 9376, 5237, 868, 9640, 8470, 1109, 1387, 9323, 4687, 4288, 616, 4207, 978, 1498, 5207, 5457, 8836, 3435, 7936, 8526, 7267, 2718, 2362, 4729, 3555, 8230, 1181, 8139, 8643, 9713, 9981, 5771, 3818, 4710, 4540, 8573, 7284, 7687, 352, 1779, 6203], dtype=np.int32)
_FL = np.array([5, 0, 0, 4, 3, 4, 1, 2, 4, 7, 4, 4, 0, 7, 1, 4, 1, 6, 1, 3, 7, 6, 2, 4, 2, 4, 6, 6, 2, 1, 2, 6, 4, 3, 3, 0, 2, 0, 2, 6, 2, 3, 6, 7, 1, 4, 1, 2, 4, 1, 7, 4, 5, 4, 0, 6, 1, 4, 4, 0, 5, 7, 7, 1, 0, 3, 5, 6, 4, 6, 4, 2, 4, 0, 2, 7, 1, 1, 0, 6, 2, 7, 1, 2, 5, 6, 5, 7, 1, 2, 1, 5, 3, 1, 1, 0, 6, 2, 7, 6, 4, 7, 4, 4, 1, 5, 7, 6, 1, 0, 4, 3, 6, 1, 6, 2, 4, 0, 1, 1, 3, 2, 1, 1, 7, 2, 4, 1, 2, 7, 3, 4, 2, 7, 2, 6, 5, 2, 6, 7, 3, 0, 2, 5, 3, 3, 0, 5, 0, 0, 6, 4, 4, 1, 6, 1, 7, 1, 3, 4], dtype=np.int32)
_RI = np.array([6417, 669, 5307, 7936, 1565, 2514, 5651, 7465, 4562, 3971, 7945, 1207, 2520, 7611, 318, 5618, 9724, 5746, 9389, 1637, 7653, 8372, 167, 4090, 3013, 357, 693, 5668, 8183, 6997, 7333, 2566, 4646, 2636, 9480, 5752, 157, 9721, 2584, 4732, 254, 1994, 2550, 3224, 7391, 1895, 4131, 426, 7330, 6961, 4319, 9821, 6380, 8771, 3323, 5968, 6462, 6015, 7178, 3430, 6732, 3417, 8879, 8842, 8634, 7623, 1348, 3497, 4026, 6147, 3103, 6515, 8582, 6590, 7215, 4145, 3291, 2965, 3982, 237, 7633, 7539, 7552, 5711, 5890, 682, 5909, 2349, 8126, 3965, 5093, 4957, 6059, 9744, 8061, 8964, 6338, 5058, 6625, 5833, 4193, 1044, 3311, 6274, 6147, 7947, 7617, 4690, 9728, 3167, 8343, 6996, 6656, 5584, 9381, 1412, 3634, 917, 1671, 6858, 5240, 1819, 4888, 5800, 7807, 8741, 3400, 242, 3307, 7869, 6945, 5022, 242, 3954, 699, 6119, 2227, 5925, 7139, 8273, 8393, 5834, 9138, 2122, 2043, 4119, 7386, 2944, 1822, 8753, 683, 2396, 9592, 3793, 6616, 4477, 9162, 6754, 5724, 8680], dtype=np.int32)
_RL = np.array([6, 4, 1, 4, 7, 4, 0, 7, 3, 6, 2, 4, 3, 4, 5, 1, 5, 2, 2, 1, 1, 6, 4, 2, 1, 1, 4, 5, 0, 7, 7, 7, 2, 0, 6, 0, 1, 2, 7, 0, 0, 0, 3, 7, 3, 3, 7, 4, 0, 3, 7, 2, 5, 2, 5, 2, 7, 4, 1, 0, 5, 7, 3, 3, 4, 5, 0, 0, 0, 1, 7, 6, 5, 5, 5, 3, 0, 7, 1, 1, 3, 3, 1, 1, 7, 6, 5, 1, 7, 3, 0, 4, 3, 7, 4, 5, 0, 6, 7, 2, 5, 5, 4, 5, 0, 3, 5, 2, 4, 1, 3, 4, 1, 0, 0, 5, 6, 6, 7, 2, 6, 5, 6, 7, 4, 4, 5, 4, 7, 7, 3, 0, 5, 3, 5, 5, 5, 7, 7, 6, 4, 6, 0, 5, 0, 1, 7, 4, 4, 1, 1, 6, 5, 2, 5, 5, 1, 7, 3, 5], dtype=np.int32)


def _norm_caps(v, bcap):
    ss = jnp.dot(v * v, bcap, preferred_element_type=jnp.float32)
    return v / jnp.maximum(jnp.sqrt(ss), 1e-12)


def _pre_body(x_ref, w_ref, b_ref, fcw_ref, fcb_ref, bcap_ref,
              h0_ref, x1_ref):
    bcap = bcap_ref[...]
    h = jnp.dot(x_ref[...], w_ref[...], preferred_element_type=jnp.float32)
    h = jnp.maximum(h + b_ref[...], 0.0)
    h0_ref[...] = h
    xs = _norm_caps(h, bcap)
    t = jnp.dot(xs, fcw_ref[...], preferred_element_type=jnp.float32)
    t = jnp.maximum(t + fcb_ref[...], 0.0)
    x1_ref[...] = _norm_caps(t, bcap)


def _pre_call(x, w, b, fcw_bd, fcb, bcap):
    return pl.pallas_call(
        _pre_body,
        grid=(N // _BLK_PRE,),
        in_specs=[
            pl.BlockSpec((_BLK_PRE, NFEAT), lambda i: (i, 0)),
            pl.BlockSpec((NFEAT, D), lambda i: (0, 0)),
            pl.BlockSpec((1, D), lambda i: (0, 0)),
            pl.BlockSpec((D, D), lambda i: (0, 0)),
            pl.BlockSpec((1, D), lambda i: (0, 0)),
            pl.BlockSpec((D, D), lambda i: (0, 0)),
        ],
        out_specs=[
            pl.BlockSpec((_BLK_PRE, D), lambda i: (i, 0)),
            pl.BlockSpec((_BLK_PRE, D), lambda i: (i, 0)),
        ],
        out_shape=[
            jax.ShapeDtypeStruct((N, D), jnp.float32),
            jax.ShapeDtypeStruct((N, D), jnp.float32),
        ],
    )(x, w, b, fcw_bd, fcb, bcap)


def _sc_gather(xsrc, idx_flat):
    mesh = plsc.VectorSubcoreMesh(core_axis_name="c", subcore_axis_name="s")
    total = idx_flat.shape[1]
    win = 128 if total % 128 == 0 else total
    width = xsrc.shape[1]

    @pl.kernel(out_type=jax.ShapeDtypeStruct((total, width), xsrc.dtype),
               mesh=mesh)
    def gk(x_hbm, i_hbm, o_hbm):
        def body(i_vmem, o_vmem):
            pltpu.sync_copy(x_hbm.at[i_vmem.at[0]], o_vmem)

        pltpu.emit_pipeline(
            body,
            grid=(total // win,),
            in_specs=[pl.BlockSpec((1, win), index_map=lambda i: (0, i))],
            out_specs=[pl.BlockSpec((win, width), index_map=lambda i: (i, 0))],
            core_axis_name=("c", "s"),
            dimension_semantics=(pltpu.PARALLEL,),
        )(i_hbm, o_hbm)

    return gk(xsrc, idx_flat)


def _route_iters(zs, u0, g, bcap):
    u = u0
    ub = u0
    one_m = 1.0 / DEG
    one_k = 1.0 / KCAPS

    def tree_sum(vals):
        while len(vals) > 1:
            vals = [a + b for a, b in zip(vals[::2], vals[1::2])]
        return vals[0]

    for it in range(ROUTIT):
        if it == 0:
            w0 = g * one_m + (1.0 - g) * one_k
            u = tree_sum(list(zs)) * w0 + ub
        else:
            s = None
            a1 = None
            a2 = None
            for m in range(DEG):
                p = jnp.dot(zs[m] * u, bcap,
                            preferred_element_type=jnp.float32)
                e = jnp.exp(p)
                ts = jnp.sum(e, axis=1, keepdims=True) * (1.0 / DD)
                ze = zs[m] * e
                zet = ze / ts
                s = e if s is None else s + e
                a1 = ze if a1 is None else a1 + ze
                a2 = zet if a2 is None else a2 + zet
            u = (g * a1 / s + (1.0 - g) * a2) + ub
        ub = u
        if it < ROUTIT - 1:
            u = _norm_caps(u, bcap)
    return u


def _route1_body(z_ref, x_ref, g_ref, bcap_ref, buf_ref, out_ref):
    del buf_ref
    bcap = bcap_ref[...]
    zs = [z_ref[m] for m in range(DEG)]
    u = _route_iters(zs, x_ref[...], g_ref[...], bcap)
    out_ref[...] = _norm_caps(jnp.maximum(u, 0.0), bcap)


def _route2_body(z_ref, x_ref, g_ref, bcap_ref, mlpw_ref, mlpb_ref,
                 bufh_ref, buflp_ref, h_ref, logp_ref):
    del bufh_ref, buflp_ref
    bcap = bcap_ref[...]
    zs = [z_ref[m] for m in range(DEG)]
    u = _route_iters(zs, x_ref[...], g_ref[...], bcap)
    h_ref[...] = u
    logit = jnp.dot(u, mlpw_ref[...], preferred_element_type=jnp.float32)
    logit = logit + mlpb_ref[...]
    mx = jnp.max(logit, axis=1, keepdims=True)
    lse = jnp.log(jnp.sum(jnp.exp(logit - mx), axis=1, keepdims=True)) + mx
    logp_ref[...] = logit - lse


def _route1_call(z, x1, g, bcap, buf, off):
    nblk = z.shape[1] // _BLK_RT
    return pl.pallas_call(
        _route1_body,
        grid=(nblk,),
        in_specs=[
            pl.BlockSpec((DEG, _BLK_RT, D), lambda i: (0, i, 0)),
            pl.BlockSpec((_BLK_RT, D), lambda i: (i + off, 0)),
            pl.BlockSpec((1, D), lambda i: (0, 0)),
            pl.BlockSpec((D, D), lambda i: (0, 0)),
            pl.BlockSpec(memory_space=pltpu.MemorySpace.HBM),
        ],
        out_specs=pl.BlockSpec((_BLK_RT, D), lambda i: (i + off, 0)),
        out_shape=jax.ShapeDtypeStruct((N, D), jnp.float32),
        input_output_aliases={4: 0},
    )(z, x1, g, bcap, buf)


def _route2_call(z, x2, g, bcap, mlpw, mlpb, bufh, buflp, off):
    nblk = z.shape[1] // _BLK_RT
    return pl.pallas_call(
        _route2_body,
        grid=(nblk,),
        in_specs=[
            pl.BlockSpec((DEG, _BLK_RT, D), lambda i: (0, i, 0)),
            pl.BlockSpec((_BLK_RT, D), lambda i: (i + off, 0)),
            pl.BlockSpec((1, D), lambda i: (0, 0)),
            pl.BlockSpec((D, D), lambda i: (0, 0)),
            pl.BlockSpec((D, NCLASS), lambda i: (0, 0)),
            pl.BlockSpec((1, NCLASS), lambda i: (0, 0)),
            pl.BlockSpec(memory_space=pltpu.MemorySpace.HBM),
            pl.BlockSpec(memory_space=pltpu.MemorySpace.HBM),
        ],
        out_specs=[
            pl.BlockSpec((_BLK_RT, D), lambda i: (i + off, 0)),
            pl.BlockSpec((_BLK_RT, NCLASS), lambda i: (i + off, 0)),
        ],
        out_shape=[
            jax.ShapeDtypeStruct((N, D), jnp.float32),
            jax.ShapeDtypeStruct((N, NCLASS), jnp.float32),
        ],
        input_output_aliases={6: 0, 7: 1},
    )(z, x2, g, bcap, mlpw, mlpb, bufh, buflp)


def _softplus(v):
    return jnp.maximum(v, 0.0) + jnp.log(1.0 + jnp.exp(-jnp.abs(v)))


def _disc_body(fake_ref, real_ref, fl_ref, rl_ref, fsel_ref, dw_ref, db_ref,
               c1w_ref, c1b_ref, c2w_ref, c2b_ref, g_ref, d_ref):
    def pick(rows, lab):
        ci = jax.lax.shift_right_logical(
            jax.lax.broadcasted_iota(jnp.int32, rows.shape, 1), 4)
        masked = jnp.where(ci == lab, rows, 0.0)
        return jnp.dot(masked, fsel_ref[...],
                       preferred_element_type=jnp.float32)

    def disc(sv):
        t = jnp.dot(sv, dw_ref[...], preferred_element_type=jnp.float32)
        t = jnp.maximum(t + db_ref[...], 0.0)
        lg1 = jnp.dot(t, c1w_ref[...],
                      preferred_element_type=jnp.float32) + c1b_ref[...]
        lg2 = jnp.dot(t, c2w_ref[...],
                      preferred_element_type=jnp.float32) + c2b_ref[...]
        return lg1, lg2

    def clsloss(lg, lb):
        mx = jnp.max(lg, axis=1, keepdims=True)
        lse = jnp.log(jnp.sum(jnp.exp(lg - mx), axis=1, keepdims=True)) + mx
        oh = jax.lax.broadcasted_iota(jnp.int32, lg.shape, 1) == lb
        sel = jnp.sum(jnp.where(oh, lg, 0.0), axis=1, keepdims=True)
        return jnp.mean(lse - sel)

    d_fake, prob_fake = disc(pick(fake_ref[...], fl_ref[...]))
    d_real, prob_real = disc(pick(real_ref[...], rl_ref[...]))
    g_total = jnp.mean(_softplus(-d_fake)) + clsloss(prob_real, rl_ref[...])
    d_total = (jnp.mean(_softplus(-d_real)) + jnp.mean(_softplus(d_fake))
               + clsloss(prob_fake, fl_ref[...]))
    g_ref[...] = g_total.reshape(1, 1)
    d_ref[...] = d_total.reshape(1, 1)


def _disc_call(fake, real, fl, rl, fsel, dw, db, c1w, c1b, c2w, c2b):
    def full(s):
        return pl.BlockSpec(s, lambda: (0,) * len(s))
    return pl.pallas_call(
        _disc_body,
        in_specs=[
            full((NUM_SAMP, D)), full((NUM_SAMP, D)),
            full((NUM_SAMP, 1)), full((NUM_SAMP, 1)),
            full((D, DD)),
            full((DD, KCAPS)), full((1, KCAPS)),
            full((KCAPS, 1)), full((1, 1)),
            full((KCAPS, KCAPS)), full((1, KCAPS)),
        ],
        out_specs=[full((1, 1)), full((1, 1))],
        out_shape=[
            jax.ShapeDtypeStruct((1, 1), jnp.float32),
            jax.ShapeDtypeStruct((1, 1), jnp.float32),
        ],
    )(fake, real, fl, rl, fsel, dw, db, c1w, c1b, c2w, c2b)


def kernel(x, nb, params):
    bcap = jnp.asarray(_B_CAP)

    fcw = params['fc_W']
    fcw_bd = jnp.zeros((D, D), jnp.float32)
    for k in range(KCAPS):
        fcw_bd = fcw_bd.at[k * DD:(k + 1) * DD, k * DD:(k + 1) * DD].set(fcw[k])
    fcb = params['fc_b'].reshape(1, D)

    gate = jax.nn.sigmoid(params['param']).reshape(1, 1)
    g_vec = jnp.broadcast_to(gate, (1, D)).astype(jnp.float32)

    h0, x1 = _pre_call(x, params['pca_W'], params['pca_b'].reshape(1, D),
                       fcw_bd, fcb, bcap)

    nchunk = 5
    chn = N // nchunk
    idx_mm = nb.T.astype(jnp.int32)

    nblk_per_chunk = chn // _BLK_RT

    def chunk_idx(c):
        return idx_mm[:, c * chn:(c + 1) * chn].reshape(1, DEG * chn)

    x2 = jnp.zeros((N, D), jnp.float32)
    for c in range(nchunk):
        zc = _sc_gather(x1, chunk_idx(c)).reshape(DEG, chn, D)
        x2 = _route1_call(zc, x1, g_vec, bcap, x2, c * nblk_per_chunk)

    fi_pad = jnp.asarray(np.pad(_FI, (0, 256 - NUM_SAMP)).reshape(1, 256))
    ri_pad = jnp.asarray(np.pad(_RI, (0, 256 - NUM_SAMP)).reshape(1, 256))
    fake_rows = _sc_gather(h0, fi_pad)[:NUM_SAMP]

    h = jnp.zeros((N, D), jnp.float32)
    logp = jnp.zeros((N, NCLASS), jnp.float32)
    for c in range(nchunk):
        zc = _sc_gather(x2, chunk_idx(c)).reshape(DEG, chn, D)
        h, logp = _route2_call(zc, x2, g_vec, bcap, params['mlp_W'],
                               params['mlp_b'].reshape(1, NCLASS),
                               h, logp, c * nblk_per_chunk)

    real_rows = _sc_gather(h, ri_pad)[:NUM_SAMP]

    g_t, d_t = _disc_call(
        fake_rows, real_rows,
        jnp.asarray(_FL).reshape(NUM_SAMP, 1),
        jnp.asarray(_RL).reshape(NUM_SAMP, 1),
        jnp.asarray(_F_SEL),
        params['disc_W'], params['disc_b'].reshape(1, KCAPS),
        params['cls1_W'], params['cls1_b'].reshape(1, 1),
        params['cls2_W'], params['cls2_b'].reshape(1, KCAPS))

    return logp, g_t.reshape(()), d_t.reshape(()), h
